# baseline (speedup 1.0000x reference)
import functools

import jax
import jax.numpy as jnp
from jax.experimental import pallas as pl
from jax.experimental.pallas import tpu as pltpu


def _mlp_kernel(x_ref, wfc_ref, bfc_ref, wg1_ref, bg1_ref, h1_ref):
    h = jnp.dot(x_ref[...], wfc_ref[...], preferred_element_type=jnp.float32)
    h = jnp.maximum(h + bfc_ref[...], 0.0)
    h1_ref[...] = (
        jnp.dot(h, wg1_ref[...], preferred_element_type=jnp.float32)
        + bg1_ref[...]
    )


def _agg_lin_kernel(adj_ref, h_ref, w_ref, b_ref, out_ref):
    t = jnp.dot(adj_ref[...], h_ref[...], preferred_element_type=jnp.float32)
    t = jnp.maximum(t, 0.0)
    out_ref[...] = (
        jnp.dot(t, w_ref[...], preferred_element_type=jnp.float32) + b_ref[...]
    )


def _agg_relu_kernel(adj_ref, h_ref, out_ref):
    t = jnp.dot(adj_ref[...], h_ref[...], preferred_element_type=jnp.float32)
    out_ref[...] = jnp.maximum(t, 0.0)


def _pick_bm(n):
    for bm in (512, 400, 256, 200, 128, 80, 40, 16, 8):
        if n % bm == 0:
            return bm
    return n


def kernel(x, adj, W_fc, b_fc, W_g1, b_g1, W_g2, b_g2):
    n, in_ft = x.shape
    h1w = W_g1.shape[1]
    outw = W_g2.shape[1]
    b_fc2 = b_fc.reshape(1, -1)
    b_g12 = b_g1.reshape(1, -1)
    b_g22 = b_g2.reshape(1, -1)

    h1 = pl.pallas_call(
        _mlp_kernel,
        out_shape=jax.ShapeDtypeStruct((n, h1w), jnp.float32),
    )(x, W_fc, b_fc2, W_g1, b_g12)

    bm = _pick_bm(n)
    grid = n // bm

    full = lambda shape: pl.BlockSpec(shape, lambda i: (0, 0))
    rowblk = lambda w: pl.BlockSpec((bm, w), lambda i: (i, 0))

    h2 = pl.pallas_call(
        _agg_lin_kernel,
        grid=(grid,),
        in_specs=[
            rowblk(n),
            full((n, h1w)),
            full(W_g2.shape),
            full(b_g22.shape),
        ],
        out_specs=rowblk(outw),
        out_shape=jax.ShapeDtypeStruct((n, outw), jnp.float32),
    )(adj, h1, W_g2, b_g22)

    out = pl.pallas_call(
        _agg_relu_kernel,
        grid=(grid,),
        in_specs=[rowblk(n), full((n, outw))],
        out_specs=rowblk(outw),
        out_shape=jax.ShapeDtypeStruct((n, outw), jnp.float32),
    )(adj, h2)
    return out


# speedup vs baseline: 1.0455x; 1.0455x over previous
import functools

import jax
import jax.numpy as jnp
from jax.experimental import pallas as pl
from jax.experimental.pallas import tpu as pltpu


def _fused_kernel(
    x_ref,
    adj_ref,
    wfc_ref,
    bfc_ref,
    wg1_ref,
    bg1_ref,
    wg2_ref,
    bg2_ref,
    out_ref,
    h1_ref,
    h2_ref,
    *,
    nblk,
    bm,
):
    i = pl.program_id(0)

    @pl.when(i == 0)
    def _():
        h = jnp.dot(x_ref[...], wfc_ref[...], preferred_element_type=jnp.float32)
        h = jnp.maximum(h + bfc_ref[...], 0.0)
        h1_ref[...] = (
            jnp.dot(h, wg1_ref[...], preferred_element_type=jnp.float32)
            + bg1_ref[...]
        )

    @pl.when((i >= 1) & (i <= nblk))
    def _():
        t = jnp.dot(adj_ref[...], h1_ref[...], preferred_element_type=jnp.float32)
        t = jnp.maximum(t, 0.0)
        h2_ref[pl.ds((i - 1) * bm, bm), :] = (
            jnp.dot(t, wg2_ref[...], preferred_element_type=jnp.float32)
            + bg2_ref[...]
        )

    @pl.when(i > nblk)
    def _():
        t = jnp.dot(adj_ref[...], h2_ref[...], preferred_element_type=jnp.float32)
        out_ref[...] = jnp.maximum(t, 0.0)


def _pick_bm(n):
    for bm in (400, 256, 200, 128, 80, 40, 16, 8):
        if n % bm == 0:
            return bm
    return n


def kernel(x, adj, W_fc, b_fc, W_g1, b_g1, W_g2, b_g2):
    n, in_ft = x.shape
    h1w = W_g1.shape[1]
    outw = W_g2.shape[1]
    b_fc2 = b_fc.reshape(1, -1)
    b_g12 = b_g1.reshape(1, -1)
    b_g22 = b_g2.reshape(1, -1)

    bm = _pick_bm(n)
    nblk = n // bm

    full = lambda shape: pl.BlockSpec(shape, lambda i: (0, 0))

    out = pl.pallas_call(
        functools.partial(_fused_kernel, nblk=nblk, bm=bm),
        grid=(2 * nblk + 1,),
        in_specs=[
            full((n, in_ft)),
            pl.BlockSpec((bm, n), lambda i: ((jnp.maximum(i, 1) - 1) % nblk, 0)),
            full(W_fc.shape),
            full(b_fc2.shape),
            full(W_g1.shape),
            full(b_g12.shape),
            full(W_g2.shape),
            full(b_g22.shape),
        ],
        out_specs=pl.BlockSpec(
            (bm, outw), lambda i: (jnp.maximum(i - (nblk + 1), 0), 0)
        ),
        out_shape=jax.ShapeDtypeStruct((n, outw), jnp.float32),
        scratch_shapes=[
            pltpu.VMEM((n, h1w), jnp.float32),
            pltpu.VMEM((n, outw), jnp.float32),
        ],
    )(x, adj, W_fc, b_fc2, W_g1, b_g12, W_g2, b_g22)
    return out
